# Initial kernel scaffold; baseline (speedup 1.0000x reference)
import jax
import jax.numpy as jnp
from jax.experimental import pallas as pl


def kernel(x, table):
    raise NotImplementedError("write your pallas kernel here")


# speedup vs baseline: 3.4284x; 3.4284x over previous
import jax
import jax.numpy as jnp
from jax.experimental import pallas as pl


_BL = 512


def _copy_body(t_ref, o_ref):
    o_ref[0] = t_ref[...]


def kernel(x, table):
    B, L, D = x.shape
    n_l = L // _BL
    out = pl.pallas_call(
        _copy_body,
        grid=(n_l, B),
        in_specs=[pl.BlockSpec((_BL, D), lambda l, b: (l, 0))],
        out_specs=pl.BlockSpec((1, _BL, D), lambda l, b: (b, l, 0)),
        out_shape=jax.ShapeDtypeStruct((B, L, D), table.dtype),
    )(table)
    return out


# speedup vs baseline: 4.2047x; 1.2264x over previous
import jax
import jax.numpy as jnp
from jax.experimental import pallas as pl


_BL = 1024


def _copy_body(t_ref, o_ref):
    o_ref[0] = t_ref[...]


def kernel(x, table):
    B, L, D = x.shape
    n_l = L // _BL
    out = pl.pallas_call(
        _copy_body,
        grid=(n_l, B),
        in_specs=[pl.BlockSpec((_BL, D), lambda l, b: (l, 0))],
        out_specs=pl.BlockSpec((1, _BL, D), lambda l, b: (b, l, 0)),
        out_shape=jax.ShapeDtypeStruct((B, L, D), table.dtype),
    )(table)
    return out


# speedup vs baseline: 4.6464x; 1.1050x over previous
import jax
import jax.numpy as jnp
from jax.experimental import pallas as pl


_BL = 2048


def _copy_body(t_ref, o_ref):
    o_ref[0] = t_ref[...]


def kernel(x, table):
    B, L, D = x.shape
    n_l = L // _BL
    out = pl.pallas_call(
        _copy_body,
        grid=(n_l, B),
        in_specs=[pl.BlockSpec((_BL, D), lambda l, b: (l, 0))],
        out_specs=pl.BlockSpec((1, _BL, D), lambda l, b: (b, l, 0)),
        out_shape=jax.ShapeDtypeStruct((B, L, D), table.dtype),
    )(table)
    return out
